# baseline (speedup 1.0000x reference)
import functools

import jax
import jax.numpy as jnp
from jax import lax
from jax.experimental import pallas as pl
from jax.experimental.pallas import tpu as pltpu
from jax.experimental.pallas import tpu_sc as plsc

_V = 1000
_C = 1000
_CP = 1024
_CB = 896
_CT = _C - _CB
_N = 1024 * 50
_NC, _NS = 2, 16
_NW = _NC * _NS
_PW = _N // _NW
_R = 40
_NCH = _PW // _R
_EG = 80
_NEG = _PW // _EG
_SBR = 1024
_SNB = _N // _SBR


def _lse_body(tab_ref, out_ref):
    x = tab_ref[...]
    m = jnp.max(x, axis=1, keepdims=True)
    s = jnp.sum(jnp.exp(x - m), axis=1, keepdims=True)
    out_ref[...] = m + jnp.log(s)


def _row_lse(table):
    out = pl.pallas_call(
        _lse_body,
        out_shape=jax.ShapeDtypeStruct((_V, 1), jnp.float32),
    )(table)
    return out.reshape(_V)


_sc_mesh = plsc.VectorSubcoreMesh(core_axis_name="c", subcore_axis_name="s")


@functools.partial(
    pl.kernel,
    out_type=(
        jax.ShapeDtypeStruct((_N, _C), jnp.float32),
        jax.ShapeDtypeStruct((_N, 128), jnp.float32),
        jax.ShapeDtypeStruct((_NW * 16,), jnp.float32),
    ),
    mesh=_sc_mesh,
    scratch_types=[
        pltpu.VMEM((_PW,), jnp.int32),
        pltpu.VMEM((_PW,), jnp.int32),
        pltpu.VMEM((_PW,), jnp.int32),
        pltpu.VMEM((_R, _CP), jnp.float32),
        pltpu.VMEM((_R, _CP), jnp.float32),
        pltpu.VMEM((_PW,), jnp.float32),
        pltpu.VMEM((_PW,), jnp.float32),
        pltpu.VMEM((16,), jnp.float32),
        pltpu.SemaphoreType.DMA,
        pltpu.SemaphoreType.DMA,
        pltpu.SemaphoreType.DMA,
        pltpu.SemaphoreType.DMA,
        pltpu.SemaphoreType.DMA,
    ],
)
def _sc_gather_loss(table, tflat, lse, ctx, tgt,
                    logits, tails, partials,
                    ctx_v, tgt_v, fidx_v, rows_v0, rows_v1, picked_v, lseg_v,
                    acc_v, gsem0, gsem1, csem0, csem1, esem):
    wid = lax.axis_index("s") * _NC + lax.axis_index("c")
    base = wid * _PW

    pltpu.sync_copy(ctx.at[pl.ds(base, _PW)], ctx_v)
    pltpu.sync_copy(tgt.at[pl.ds(base, _PW)], tgt_v)

    def _fi(j, _):
        c = ctx_v[pl.ds(j * 16, 16)]
        t = tgt_v[pl.ds(j * 16, 16)]
        fidx_v[pl.ds(j * 16, 16)] = c * _C + t
        return 0
    lax.fori_loop(0, _PW // 16, _fi, 0)

    def _eg(k, _):
        o = k * _EG
        pltpu.make_async_copy(
            tflat.at[fidx_v.at[pl.ds(o, _EG)]],
            picked_v.at[pl.ds(o, _EG)], esem).start()
        pltpu.make_async_copy(
            lse.at[ctx_v.at[pl.ds(o, _EG)]],
            lseg_v.at[pl.ds(o, _EG)], esem).start()
        return 0
    lax.fori_loop(0, _NEG, _eg, 0)

    bufs = (rows_v0, rows_v1)
    gsems = (gsem0, gsem1)
    csems = (csem0, csem1)

    def _g(k, b):
        pltpu.make_async_copy(
            table.at[ctx_v.at[pl.ds(k * _R, _R)]], bufs[b], gsems[b]).start()

    def _gw(k, b):
        pltpu.make_async_copy(
            table.at[ctx_v.at[pl.ds(k * _R, _R)]], bufs[b], gsems[b]).wait()

    def _cdesc(k, b):
        r0 = base + k * _R
        body = pltpu.make_async_copy(
            bufs[b].at[:, pl.ds(0, _CB)],
            logits.at[pl.ds(r0, _R), pl.ds(0, _CB)], csems[b])
        tail = pltpu.make_async_copy(
            bufs[b].at[:, pl.ds(_CB, 128)],
            tails.at[pl.ds(r0, _R)], csems[b])
        return body, tail

    def _c(k, b):
        body, tail = _cdesc(k, b)
        body.start()
        tail.start()

    def _cw(k, b):
        body, tail = _cdesc(k, b)
        body.wait()
        tail.wait()

    _g(0, 0)
    _gw(0, 0)
    _g(1, 1)
    _c(0, 0)

    def _pipe(g, _):
        for (dk, b) in ((-1, 1), (0, 0)):
            k = 2 * g + dk
            o = 1 - b
            _gw(k, b)
            _cw(k - 1, o)
            _g(k + 1, o)
            _c(k, b)
        return 0
    lax.fori_loop(1, _NCH // 2, _pipe, 0)

    kl = _NCH - 1
    _gw(kl, 1)
    _cw(kl - 1, 0)
    _c(kl, 1)
    _cw(kl, 1)

    def _ed(k, _):
        o = k * _EG
        pltpu.make_async_copy(
            tflat.at[fidx_v.at[pl.ds(o, _EG)]],
            picked_v.at[pl.ds(o, _EG)], esem).wait()
        pltpu.make_async_copy(
            lse.at[ctx_v.at[pl.ds(o, _EG)]],
            lseg_v.at[pl.ds(o, _EG)], esem).wait()
        return 0
    lax.fori_loop(0, _NEG, _ed, 0)

    def _ls(j, acc):
        return acc + (lseg_v[pl.ds(j * 16, 16)] - picked_v[pl.ds(j * 16, 16)])
    acc = lax.fori_loop(0, _PW // 16, _ls, jnp.zeros((16,), jnp.float32))
    acc_v[...] = acc * (1.0 / _N)
    pltpu.sync_copy(acc_v, partials.at[pl.ds(wid * 16, 16)])


def _splice_body(tail_blk, x_ref, out_ref, vbuf, sem_out):
    i = pl.program_id(0)
    vbuf[:, pl.ds(_CB, _CT)] = tail_blk[:, pl.ds(0, _CT)]
    cout = pltpu.make_async_copy(
        vbuf.at[:, pl.ds(_CB, _CT)],
        out_ref.at[pl.ds(i * _SBR, _SBR), pl.ds(_CB, _CT)], sem_out)
    cout.start()
    cout.wait()


def _splice(tails, x):
    return pl.pallas_call(
        _splice_body,
        grid=(_SNB,),
        in_specs=[pl.BlockSpec((_SBR, 128), lambda i: (i, 0)),
                  pl.BlockSpec(memory_space=pl.ANY)],
        out_specs=pl.BlockSpec(memory_space=pl.ANY),
        out_shape=jax.ShapeDtypeStruct((_N, _C), jnp.float32),
        scratch_shapes=[pltpu.VMEM((_SBR, _C), jnp.float32),
                        pltpu.SemaphoreType.DMA],
        input_output_aliases={1: 0},
    )(tails, x)


def kernel(contexts, targets, token_embedding_table):
    table = token_embedding_table
    ctx = contexts.reshape(_N)
    tgt = targets.reshape(_N)
    lse = _row_lse(table)
    table_p = jnp.pad(table, ((0, 0), (0, _CP - _C)))
    logits_b, tails, partials = _sc_gather_loss(
        table_p, table.reshape(_V * _C), lse, ctx, tgt)
    logits = _splice(tails, logits_b)
    loss = jnp.sum(partials)
    return (logits, loss)


# speedup vs baseline: 1.2617x; 1.2617x over previous
import functools

import jax
import jax.numpy as jnp
from jax import lax
from jax.experimental import pallas as pl
from jax.experimental.pallas import tpu as pltpu
from jax.experimental.pallas import tpu_sc as plsc

_V = 1000
_C = 1000
_CP = 1024
_N = 1024 * 50
_NC, _NS = 2, 16
_NW = _NC * _NS
_PW = _N // _NW
_R = 32
_NCH = _PW // _R
_EG = 80
_NEG = _PW // _EG


def _lse_body(tab_ref, out_ref):
    x = tab_ref[...]
    m = jnp.max(x, axis=1, keepdims=True)
    s = jnp.sum(jnp.exp(x - m), axis=1, keepdims=True)
    out_ref[...] = m + jnp.log(s)


def _row_lse(table):
    out = pl.pallas_call(
        _lse_body,
        out_shape=jax.ShapeDtypeStruct((_V, 1), jnp.float32),
    )(table)
    return out.reshape(_V)


_sc_mesh = plsc.VectorSubcoreMesh(core_axis_name="c", subcore_axis_name="s")


@functools.partial(
    pl.kernel,
    out_type=(
        jax.ShapeDtypeStruct((_N, _CP), jnp.float32),
        jax.ShapeDtypeStruct((_NW * 16,), jnp.float32),
    ),
    mesh=_sc_mesh,
    scratch_types=[
        pltpu.VMEM((_PW,), jnp.int32),
        pltpu.VMEM((_PW,), jnp.int32),
        pltpu.VMEM((_PW,), jnp.int32),
        pltpu.VMEM((_R, _CP), jnp.float32),
        pltpu.VMEM((_R, _CP), jnp.float32),
        pltpu.VMEM((_R, _CP), jnp.float32),
        pltpu.VMEM((_PW,), jnp.float32),
        pltpu.VMEM((_PW,), jnp.float32),
        pltpu.VMEM((16,), jnp.float32),
        pltpu.SemaphoreType.DMA,
        pltpu.SemaphoreType.DMA,
        pltpu.SemaphoreType.DMA,
        pltpu.SemaphoreType.DMA,
        pltpu.SemaphoreType.DMA,
        pltpu.SemaphoreType.DMA,
        pltpu.SemaphoreType.DMA,
    ],
)
def _sc_gather_loss(table, tflat, lse, ctx, tgt,
                    logits, partials,
                    ctx_v, tgt_v, fidx_v, rows_v0, rows_v1, rows_v2,
                    picked_v, lseg_v, acc_v,
                    gsem0, gsem1, gsem2, csem0, csem1, csem2, esem):
    wid = lax.axis_index("s") * _NC + lax.axis_index("c")
    base = wid * _PW

    pltpu.sync_copy(ctx.at[pl.ds(base, _PW)], ctx_v)
    pltpu.sync_copy(tgt.at[pl.ds(base, _PW)], tgt_v)

    def _fi(j, _):
        c = ctx_v[pl.ds(j * 16, 16)]
        t = tgt_v[pl.ds(j * 16, 16)]
        fidx_v[pl.ds(j * 16, 16)] = c * _C + t
        return 0
    lax.fori_loop(0, _PW // 16, _fi, 0)

    def _eg(k, _):
        o = k * _EG
        pltpu.make_async_copy(
            tflat.at[fidx_v.at[pl.ds(o, _EG)]],
            picked_v.at[pl.ds(o, _EG)], esem).start()
        pltpu.make_async_copy(
            lse.at[ctx_v.at[pl.ds(o, _EG)]],
            lseg_v.at[pl.ds(o, _EG)], esem).start()
        return 0
    lax.fori_loop(0, _NEG, _eg, 0)

    bufs = (rows_v0, rows_v1, rows_v2)
    gsems = (gsem0, gsem1, gsem2)
    csems = (csem0, csem1, csem2)

    def _g(k, b):
        pltpu.make_async_copy(
            table.at[ctx_v.at[pl.ds(k * _R, _R)]], bufs[b], gsems[b]).start()

    def _gw(k, b):
        pltpu.make_async_copy(
            table.at[ctx_v.at[pl.ds(k * _R, _R)]], bufs[b], gsems[b]).wait()

    def _c(k, b):
        pltpu.make_async_copy(
            bufs[b], logits.at[pl.ds(base + k * _R, _R)], csems[b]).start()

    def _cw(k, b):
        pltpu.make_async_copy(
            bufs[b], logits.at[pl.ds(base + k * _R, _R)], csems[b]).wait()

    _g(0, 0)
    _g(1, 1)
    _gw(0, 0)
    _c(0, 0)
    _g(2, 2)
    _gw(1, 1)
    _c(1, 1)
    _cw(0, 0)
    _g(3, 0)
    _gw(2, 2)
    _c(2, 2)
    _cw(1, 1)
    _g(4, 1)

    def _pipe(gi, _):
        for b in range(3):
            k = 3 * gi + b
            bp = (b + 2) % 3
            _gw(k, b)
            _c(k, b)
            _cw(k - 1, bp)
            _g(k + 2, bp)
        return 0
    lax.fori_loop(1, (_NCH - 2) // 3, _pipe, 0)

    _gw(_NCH - 2, 0)
    _c(_NCH - 2, 0)
    _cw(_NCH - 3, 2)
    _gw(_NCH - 1, 1)
    _c(_NCH - 1, 1)
    _cw(_NCH - 2, 0)
    _cw(_NCH - 1, 1)

    def _ed(k, _):
        o = k * _EG
        pltpu.make_async_copy(
            tflat.at[fidx_v.at[pl.ds(o, _EG)]],
            picked_v.at[pl.ds(o, _EG)], esem).wait()
        pltpu.make_async_copy(
            lse.at[ctx_v.at[pl.ds(o, _EG)]],
            lseg_v.at[pl.ds(o, _EG)], esem).wait()
        return 0
    lax.fori_loop(0, _NEG, _ed, 0)

    def _ls(j, acc):
        return acc + (lseg_v[pl.ds(j * 16, 16)] - picked_v[pl.ds(j * 16, 16)])
    acc = lax.fori_loop(0, _PW // 16, _ls, jnp.zeros((16,), jnp.float32))
    acc_v[...] = acc * (1.0 / _N)
    pltpu.sync_copy(acc_v, partials.at[pl.ds(wid * 16, 16)])


def kernel(contexts, targets, token_embedding_table):
    table = token_embedding_table
    ctx = contexts.reshape(_N)
    tgt = targets.reshape(_N)
    lse = _row_lse(table)
    table_p = jnp.pad(table, ((0, 0), (0, _CP - _C)))
    logits_p, partials = _sc_gather_loss(
        table_p, table.reshape(_V * _C), lse, ctx, tgt)
    loss = jnp.sum(partials)
    return (logits_p[:, :_C], loss)
